# Initial kernel scaffold; baseline (speedup 1.0000x reference)
import jax
import jax.numpy as jnp
from jax.experimental import pallas as pl


def kernel(x, bounds, table):
    raise NotImplementedError("write your pallas kernel here")


# speedup vs baseline: 42.4673x; 42.4673x over previous
import functools

import jax
import jax.numpy as jnp
from jax import lax
from jax.experimental import pallas as pl
from jax.experimental.pallas import tpu as pltpu
from jax.experimental.pallas import tpu_sc as plsc

N_BINS = 1024
HIDDEN = 16
MIN_VAL = -3.0
MAX_VAL = 3.0
BATCH = 16384
FIELDS = 100
TOTAL = BATCH * FIELDS
NC, NS, LANES = 2, 16, 16
NW = NC * NS
PER_W = TOTAL // NW
CHUNK = 2048
STEPS = PER_W // CHUNK
INV_STEP = float(N_BINS - 2) / (MAX_VAL - MIN_VAL)

@functools.cache
def _build():
    mesh = plsc.VectorSubcoreMesh(core_axis_name="c", subcore_axis_name="s")

    @functools.partial(
        pl.kernel,
        mesh=mesh,
        out_type=jax.ShapeDtypeStruct((TOTAL, HIDDEN), jnp.float32),
        scratch_types=[
            pltpu.VMEM((N_BINS,), jnp.float32),
            pltpu.VMEM((CHUNK,), jnp.float32),
            pltpu.VMEM((CHUNK,), jnp.int32),
            pltpu.VMEM((CHUNK, HIDDEN), jnp.float32),
            pltpu.SemaphoreType.DMA,
        ],
        compiler_params=pltpu.CompilerParams(
            use_tc_tiling_on_sc=False, needs_layout_passes=False
        ),
    )
    def _sc_embed(x_hbm, bounds_hbm, table_hbm, out_hbm, bounds_v, xv, iv, rows, sem):
        wid = lax.axis_index("s") * NC + lax.axis_index("c")
        base = wid * PER_W
        pltpu.sync_copy(bounds_hbm, bounds_v)

        def step(s, carry):
            off = base + s * CHUNK
            pltpu.sync_copy(x_hbm.at[pl.ds(off, CHUNK)], xv)

            def vec(i, c):
                xx = xv[pl.ds(i * LANES, LANES)]
                t = (xx - MIN_VAL) * INV_STEP
                t = jnp.clip(t, 0.0, float(N_BINS - 1))
                g = t.astype(jnp.int32)
                g = g + jnp.where(t > g.astype(jnp.float32), 1, 0)
                bu = plsc.load_gather(bounds_v, [jnp.minimum(g, N_BINS - 2)])
                g = g + jnp.where((g < N_BINS - 1) & (bu < xx), 1, 0)
                bd = plsc.load_gather(bounds_v, [jnp.maximum(g - 1, 0)])
                g = g - jnp.where((g > 0) & (bd >= xx), 1, 0)
                iv[pl.ds(i * LANES, LANES)] = g
                return c

            lax.fori_loop(0, CHUNK // LANES, vec, 0, unroll=False)
            pltpu.async_copy(table_hbm.at[iv], rows, sem).wait()
            pltpu.sync_copy(rows, out_hbm.at[pl.ds(off, CHUNK)])
            return carry

        lax.fori_loop(0, STEPS, step, 0, unroll=False)

    return _sc_embed


def kernel(x, bounds, table):
    xf = x.reshape(TOTAL)
    bpad = jnp.concatenate([bounds, jnp.full((1,), MAX_VAL, jnp.float32)])
    out = _build()(xf, bpad, table)
    return out.reshape(BATCH, FIELDS, HIDDEN)


# speedup vs baseline: 117.3367x; 2.7630x over previous
import functools

import jax
import jax.numpy as jnp
from jax import lax
from jax.experimental import pallas as pl
from jax.experimental.pallas import tpu as pltpu
from jax.experimental.pallas import tpu_sc as plsc

N_BINS = 1024
HIDDEN = 16
MIN_VAL = -3.0
MAX_VAL = 3.0
BATCH = 16384
FIELDS = 100
NC, NS, LANES = 2, 16, 16
NW = NC * NS
ROWS_W = BATCH // NW
RBLK = 32
NBLK = ROWS_W // RBLK
WIN = (0, 16, 32, 48, 64, 80, 84)
INV_STEP = float(N_BINS - 2) / (MAX_VAL - MIN_VAL)


@functools.cache
def _build():
    mesh = plsc.VectorSubcoreMesh(core_axis_name="c", subcore_axis_name="s")

    @functools.partial(
        pl.kernel,
        mesh=mesh,
        out_type=jax.ShapeDtypeStruct((BATCH, FIELDS, HIDDEN), jnp.float32),
        scratch_types=[
            pltpu.VMEM((N_BINS - 1,), jnp.float32),
            pltpu.VMEM((RBLK, FIELDS), jnp.float32),
            pltpu.VMEM((RBLK, FIELDS), jnp.int32),
            pltpu.VMEM((RBLK, FIELDS, HIDDEN), jnp.float32),
            pltpu.SemaphoreType.DMA,
        ],
        compiler_params=pltpu.CompilerParams(
            use_tc_tiling_on_sc=False, needs_layout_passes=False
        ),
    )
    def _sc_embed(x_hbm, bounds_hbm, table_hbm, out_hbm, bounds_v, xv, iv, rv, sem):
        wid = lax.axis_index("s") * NC + lax.axis_index("c")
        base = wid * ROWS_W
        pltpu.sync_copy(bounds_hbm, bounds_v)

        def block(b, carry):
            row0 = base + b * RBLK
            pltpu.sync_copy(x_hbm.at[pl.ds(row0, RBLK)], xv)

            def row(r, c):
                for off in WIN:
                    xx = xv[r, pl.ds(off, LANES)]
                    t = jnp.clip((xx - MIN_VAL) * INV_STEP, 0.0, float(N_BINS - 1))
                    g = t.astype(jnp.int32)
                    g = g + jnp.where(t > g.astype(jnp.float32), 1, 0)
                    bu = plsc.load_gather(bounds_v, [jnp.minimum(g, N_BINS - 2)])
                    g = g + jnp.where((g < N_BINS - 1) & (bu < xx), 1, 0)
                    bd = plsc.load_gather(bounds_v, [jnp.maximum(g - 1, 0)])
                    g = g - jnp.where((g > 0) & (bd >= xx), 1, 0)
                    iv[r, pl.ds(off, LANES)] = g
                return c

            lax.fori_loop(0, RBLK, row, 0, unroll=False)

            def fire(r, c):
                pltpu.async_copy(table_hbm.at[iv.at[r]], rv.at[r], sem)
                return c

            lax.fori_loop(0, RBLK, fire, 0, unroll=False)

            def drain(r, c):
                pltpu.make_async_copy(table_hbm.at[iv.at[r]], rv.at[r], sem).wait()
                return c

            lax.fori_loop(0, RBLK, drain, 0, unroll=False)
            pltpu.sync_copy(rv, out_hbm.at[pl.ds(row0, RBLK)])
            return carry

        lax.fori_loop(0, NBLK, block, 0, unroll=False)

    return _sc_embed


def kernel(x, bounds, table):
    return _build()(x, bounds, table)


# speedup vs baseline: 214.4763x; 1.8279x over previous
import functools

import jax
import jax.numpy as jnp
from jax import lax
from jax.experimental import pallas as pl
from jax.experimental.pallas import tpu as pltpu
from jax.experimental.pallas import tpu_sc as plsc

N_BINS = 1024
HIDDEN = 16
MIN_VAL = -3.0
MAX_VAL = 3.0
BATCH = 16384
FIELDS = 100
NC, NS, LANES = 2, 16, 16
NW = NC * NS
BW = BATCH // NW
BH = 256
FB = 10
HT, HS, BT, BL = HIDDEN // 8, 8, BATCH // 128, 128
WIN = (0, 16, 32, 48, 64, 80, 84)
INV_STEP = float(N_BINS - 2) / (MAX_VAL - MIN_VAL)


@functools.cache
def _build():
    mesh = plsc.VectorSubcoreMesh(core_axis_name="c", subcore_axis_name="s")

    @functools.partial(
        pl.kernel,
        mesh=mesh,
        out_type=jax.ShapeDtypeStruct((FIELDS, HT, BT, HS, BL), jnp.float32),
        scratch_types=[
            pltpu.VMEM((N_BINS - 1,), jnp.float32),
            pltpu.VMEM((N_BINS, HIDDEN), jnp.float32),
            pltpu.VMEM((BH, FIELDS), jnp.float32),
            pltpu.VMEM((FIELDS * BH,), jnp.int32),
            pltpu.VMEM((FB, HT, BH // 128, HS, BL), jnp.float32),
        ],
        compiler_params=pltpu.CompilerParams(
            use_tc_tiling_on_sc=False, needs_layout_passes=False
        ),
    )
    def _sc_embed(x_hbm, bounds_hbm, table_hbm, out_hbm, bounds_v, table_v, xv, it, vb):
        wid = lax.axis_index("s") * NC + lax.axis_index("c")
        pltpu.sync_copy(bounds_hbm, bounds_v)
        pltpu.sync_copy(table_hbm, table_v)
        iota = lax.iota(jnp.int32, LANES)
        i256 = iota * BH
        hvecs = [jnp.full((LANES,), h, jnp.int32) for h in range(HIDDEN)]

        def half(ph, carry):
            b0 = wid * BW + ph * BH
            pltpu.sync_copy(x_hbm.at[pl.ds(b0, BH)], xv)

            def brow(b, c):
                for off in WIN:
                    xx = xv[b, pl.ds(off, LANES)]
                    t = jnp.clip((xx - MIN_VAL) * INV_STEP, 0.0, float(N_BINS - 1))
                    g = t.astype(jnp.int32)
                    g = g + jnp.where(t > g.astype(jnp.float32), 1, 0)
                    bu = plsc.load_gather(bounds_v, [jnp.minimum(g, N_BINS - 2)])
                    g = g + jnp.where((g < N_BINS - 1) & (bu < xx), 1, 0)
                    bd = plsc.load_gather(bounds_v, [jnp.maximum(g - 1, 0)])
                    g = g - jnp.where((g > 0) & (bd >= xx), 1, 0)
                    plsc.store_scatter(it, [i256 + (off * BH + b)], g)
                return c

            lax.fori_loop(0, BH, brow, 0, unroll=False)

            def fblock(fb, c):
                f0 = fb * FB

                def field(f_, c2):
                    def bwin(w, c3):
                        iv16 = plsc.load_gather(
                            it, [iota + ((f0 + f_) * BH + w * LANES)])
                        bt_ = w // 8
                        bl_ = (w % 8) * LANES
                        for h in range(HIDDEN):
                            vals = plsc.load_gather(table_v, [iv16, hvecs[h]])
                            vb[f_, h // 8, bt_, h % 8, pl.ds(bl_, LANES)] = vals
                        return c3

                    lax.fori_loop(0, BH // LANES, bwin, 0, unroll=False)
                    return c2

                lax.fori_loop(0, FB, field, 0, unroll=False)
                pltpu.sync_copy(
                    vb,
                    out_hbm.at[pl.ds(f0, FB), :, pl.ds(b0 // 128, BH // 128)],
                )
                return c

            lax.fori_loop(0, FIELDS // FB, fblock, 0, unroll=False)
            return carry

        lax.fori_loop(0, BW // BH, half, 0, unroll=False)

    return _sc_embed


def kernel(x, bounds, table):
    raw = _build()(x, bounds, table)
    out = raw.transpose(2, 4, 0, 1, 3)
    return out.reshape(BATCH, FIELDS, HIDDEN)


# speedup vs baseline: 244.9199x; 1.1419x over previous
import functools

import jax
import jax.numpy as jnp
from jax import lax
from jax.experimental import pallas as pl
from jax.experimental.pallas import tpu as pltpu
from jax.experimental.pallas import tpu_sc as plsc

N_BINS = 1024
HIDDEN = 16
MIN_VAL = -3.0
MAX_VAL = 3.0
BATCH = 16384
FIELDS = 100
NC, NS, LANES = 2, 16, 16
NW = NC * NS
BW = BATCH // NW
BH = 256
FB = 10
HT, HS, BT, BL = HIDDEN // 8, 8, BATCH // 128, 128
NPAIR = FIELDS // (2 * FB)
INV_STEP = float(N_BINS - 2) / (MAX_VAL - MIN_VAL)


@functools.cache
def _build():
    mesh = plsc.VectorSubcoreMesh(core_axis_name="c", subcore_axis_name="s")

    @functools.partial(
        pl.kernel,
        mesh=mesh,
        out_type=jax.ShapeDtypeStruct((FIELDS, HT, BT, HS, BL), jnp.float32),
        scratch_types=[
            pltpu.VMEM((N_BINS - 1,), jnp.float32),
            pltpu.VMEM((N_BINS * HIDDEN,), jnp.float32),
            pltpu.VMEM((FIELDS, BH), jnp.float32),
            pltpu.VMEM((FB, HT, BH // 128, HS, BL), jnp.float32),
            pltpu.VMEM((FB, HT, BH // 128, HS, BL), jnp.float32),
            pltpu.SemaphoreType.DMA,
            pltpu.SemaphoreType.DMA,
        ],
        compiler_params=pltpu.CompilerParams(
            use_tc_tiling_on_sc=False, needs_layout_passes=False
        ),
    )
    def _sc_embed(xt_hbm, bounds_hbm, tflat_hbm, out_hbm,
                  bounds_v, tv, xv, vb0, vb1, sem0, sem1):
        wid = lax.axis_index("s") * NC + lax.axis_index("c")
        pltpu.sync_copy(bounds_hbm, bounds_v)
        pltpu.sync_copy(tflat_hbm, tv)
        iota = lax.iota(jnp.int32, LANES)

        def half(ph, carry):
            b0 = wid * BW + ph * BH
            bt0 = b0 // 128
            pltpu.sync_copy(xt_hbm.at[:, pl.ds(b0, BH)], xv)

            def fields_into(vb, f0base):
                def field(f_, c2):
                    fa = f0base + f_

                    def bwin(w, c3):
                        xx = xv[fa, pl.ds(w * LANES, LANES)]
                        t = jnp.clip((xx - MIN_VAL) * INV_STEP,
                                     0.0, float(N_BINS - 1))
                        g = t.astype(jnp.int32)
                        g = g + jnp.where(t > g.astype(jnp.float32), 1, 0)
                        bu = plsc.load_gather(
                            bounds_v, [jnp.minimum(g, N_BINS - 2)])
                        g = g + jnp.where((g < N_BINS - 1) & (bu < xx), 1, 0)
                        bd = plsc.load_gather(
                            bounds_v, [jnp.maximum(g - 1, 0)])
                        g = g - jnp.where((g > 0) & (bd >= xx), 1, 0)
                        av = g * HIDDEN
                        bt_ = w // 8
                        bl_ = (w % 8) * LANES
                        for h in range(HIDDEN):
                            vals = plsc.load_gather(tv, [av + h])
                            vb[f_, h // 8, bt_, h % 8, pl.ds(bl_, LANES)] = vals
                        return c3

                    lax.fori_loop(0, BH // LANES, bwin, 0, unroll=True)
                    return c2

                lax.fori_loop(0, FB, field, 0, unroll=False)

            def pair(p, c):
                f0 = p * 2 * FB

                @pl.when(p > 0)
                def _():
                    pltpu.make_async_copy(
                        vb0, out_hbm.at[pl.ds(0, FB), :, pl.ds(bt0, BH // 128)],
                        sem0).wait()

                fields_into(vb0, f0)
                pltpu.async_copy(
                    vb0, out_hbm.at[pl.ds(f0, FB), :, pl.ds(bt0, BH // 128)],
                    sem0)

                @pl.when(p > 0)
                def _():
                    pltpu.make_async_copy(
                        vb1, out_hbm.at[pl.ds(0, FB), :, pl.ds(bt0, BH // 128)],
                        sem1).wait()

                fields_into(vb1, f0 + FB)
                pltpu.async_copy(
                    vb1, out_hbm.at[pl.ds(f0 + FB, FB), :,
                                    pl.ds(bt0, BH // 128)],
                    sem1)
                return c

            lax.fori_loop(0, NPAIR, pair, 0, unroll=False)
            pltpu.make_async_copy(
                vb0, out_hbm.at[pl.ds(0, FB), :, pl.ds(bt0, BH // 128)],
                sem0).wait()
            pltpu.make_async_copy(
                vb1, out_hbm.at[pl.ds(0, FB), :, pl.ds(bt0, BH // 128)],
                sem1).wait()
            return carry

        lax.fori_loop(0, BW // BH, half, 0, unroll=False)

    return _sc_embed


def kernel(x, bounds, table):
    raw = _build()(x.T, bounds, table.reshape(N_BINS * HIDDEN))
    out = raw.transpose(2, 4, 0, 1, 3)
    return out.reshape(BATCH, FIELDS, HIDDEN)


# speedup vs baseline: 617.0190x; 2.5193x over previous
import functools

import jax
import jax.numpy as jnp
from jax import lax
from jax.experimental import pallas as pl
from jax.experimental.pallas import tpu as pltpu
from jax.experimental.pallas import tpu_sc as plsc

N_BINS = 1024
HIDDEN = 16
MIN_VAL = -3.0
MAX_VAL = 3.0
BATCH = 16384
FIELDS = 100
NC, NS, LANES = 2, 16, 16
NW = NC * NS
BW = BATCH // NW
BH = 256
FB = 10
HT, HS, BT, BL = HIDDEN // 8, 8, BATCH // 128, 128
NPAIR = FIELDS // (2 * FB)
INV_STEP = float(N_BINS - 2) / (MAX_VAL - MIN_VAL)


@functools.cache
def _build():
    mesh = plsc.VectorSubcoreMesh(core_axis_name="c", subcore_axis_name="s")

    @functools.partial(
        pl.kernel,
        mesh=mesh,
        out_type=jax.ShapeDtypeStruct((FIELDS, HT, BT, HS, BL), jnp.float32),
        scratch_types=[
            pltpu.VMEM((N_BINS - 1,), jnp.float32),
            pltpu.VMEM((N_BINS * HIDDEN,), jnp.float32),
            pltpu.VMEM((FIELDS, BH), jnp.float32),
            pltpu.VMEM((FB, HT, BH // 128, HS, BL), jnp.float32),
            pltpu.VMEM((FB, HT, BH // 128, HS, BL), jnp.float32),
            pltpu.SemaphoreType.DMA,
            pltpu.SemaphoreType.DMA,
        ],
        compiler_params=pltpu.CompilerParams(
            use_tc_tiling_on_sc=False, needs_layout_passes=False
        ),
    )
    def _sc_embed(xt_hbm, bounds_hbm, tflat_hbm, out_hbm,
                  bounds_v, tv, xv, vb0, vb1, sem0, sem1):
        wid = lax.axis_index("s") * NC + lax.axis_index("c")
        pltpu.sync_copy(bounds_hbm, bounds_v)
        pltpu.sync_copy(tflat_hbm, tv)
        iota = lax.iota(jnp.int32, LANES)

        def half(ph, carry):
            b0 = wid * BW + ph * BH
            bt0 = b0 // 128
            pltpu.sync_copy(xt_hbm.at[:, pl.ds(b0, BH)], xv)

            def fields_into(vb, f0base):
                @plsc.parallel_loop(0, FB)
                def field(f_):
                    fa = f0base + f_

                    @plsc.parallel_loop(0, BH // LANES, unroll=4)
                    def bwin(w):
                        xx = xv[fa, pl.ds(w * LANES, LANES)]
                        t = jnp.clip((xx - MIN_VAL) * INV_STEP,
                                     0.0, float(N_BINS - 1))
                        g = t.astype(jnp.int32)
                        g = g + jnp.where(t > g.astype(jnp.float32), 1, 0)
                        bu = plsc.load_gather(
                            bounds_v, [jnp.minimum(g, N_BINS - 2)])
                        bd = plsc.load_gather(
                            bounds_v, [jnp.maximum(g - 1, 0)])
                        up = jnp.where((g < N_BINS - 1) & (bu < xx), 1, 0)
                        dn = jnp.where((g > 0) & (bd >= xx), 1, 0)
                        av = (g + up - dn) * HIDDEN
                        bt_ = w // 8
                        bl_ = (w % 8) * LANES
                        for h in range(HIDDEN):
                            vals = plsc.load_gather(tv, [av + h])
                            vb[f_, h // 8, bt_, h % 8, pl.ds(bl_, LANES)] = vals

            def pair(p, c):
                f0 = p * 2 * FB

                @pl.when(p > 0)
                def _():
                    pltpu.make_async_copy(
                        vb0, out_hbm.at[pl.ds(0, FB), :, pl.ds(bt0, BH // 128)],
                        sem0).wait()

                fields_into(vb0, f0)
                pltpu.async_copy(
                    vb0, out_hbm.at[pl.ds(f0, FB), :, pl.ds(bt0, BH // 128)],
                    sem0)

                @pl.when(p > 0)
                def _():
                    pltpu.make_async_copy(
                        vb1, out_hbm.at[pl.ds(0, FB), :, pl.ds(bt0, BH // 128)],
                        sem1).wait()

                fields_into(vb1, f0 + FB)
                pltpu.async_copy(
                    vb1, out_hbm.at[pl.ds(f0 + FB, FB), :,
                                    pl.ds(bt0, BH // 128)],
                    sem1)
                return c

            lax.fori_loop(0, NPAIR, pair, 0, unroll=False)
            pltpu.make_async_copy(
                vb0, out_hbm.at[pl.ds(0, FB), :, pl.ds(bt0, BH // 128)],
                sem0).wait()
            pltpu.make_async_copy(
                vb1, out_hbm.at[pl.ds(0, FB), :, pl.ds(bt0, BH // 128)],
                sem1).wait()
            return carry

        lax.fori_loop(0, BW // BH, half, 0, unroll=False)

    return _sc_embed


def kernel(x, bounds, table):
    raw = _build()(x.T, bounds, table.reshape(N_BINS * HIDDEN))
    out = raw.transpose(2, 4, 0, 1, 3)
    return out.reshape(BATCH, FIELDS, HIDDEN)


# speedup vs baseline: 1423.9337x; 2.3078x over previous
import functools

import jax
import jax.numpy as jnp
from jax import lax
from jax.experimental import pallas as pl
from jax.experimental.pallas import tpu as pltpu
from jax.experimental.pallas import tpu_sc as plsc

N_BINS = 1024
HIDDEN = 16
MIN_VAL = -3.0
MAX_VAL = 3.0
BATCH = 16384
FIELDS = 100
NC, NS, LANES = 2, 16, 16
NW = NC * NS
BW = BATCH // NW
BH = 256
FB = 10
HT, HS, BT, BL = HIDDEN // 8, 8, BATCH // 128, 128
NPAIR = FIELDS // (2 * FB)
INV_STEP = float(N_BINS - 2) / (MAX_VAL - MIN_VAL)


@functools.cache
def _build():
    mesh = plsc.VectorSubcoreMesh(core_axis_name="c", subcore_axis_name="s")

    @functools.partial(
        pl.kernel,
        mesh=mesh,
        out_type=jax.ShapeDtypeStruct((FIELDS, HT, BT, HS, BL), jnp.float32),
        scratch_types=[
            pltpu.VMEM((N_BINS - 1,), jnp.float32),
            pltpu.VMEM((N_BINS * (HIDDEN + 1),), jnp.float32),
            pltpu.VMEM((FIELDS, BH), jnp.float32),
            pltpu.VMEM((FB, HT, BH // 128, HS, BL), jnp.float32),
            pltpu.VMEM((FB, HT, BH // 128, HS, BL), jnp.float32),
            pltpu.SemaphoreType.DMA,
            pltpu.SemaphoreType.DMA,
        ],
        compiler_params=pltpu.CompilerParams(
            use_tc_tiling_on_sc=False, needs_layout_passes=False
        ),
    )
    def _sc_embed(xt_hbm, bounds_hbm, tflat_hbm, out_hbm,
                  bounds_v, tv, xv, vb0, vb1, sem0, sem1):
        wid = lax.axis_index("s") * NC + lax.axis_index("c")
        pltpu.sync_copy(bounds_hbm, bounds_v)
        pltpu.sync_copy(tflat_hbm, tv)
        iota = lax.iota(jnp.int32, LANES)

        def half(ph, carry):
            b0 = wid * BW + ph * BH
            bt0 = b0 // 128
            pltpu.sync_copy(xt_hbm.at[:, pl.ds(b0, BH)], xv)

            def fields_into(vb, f0base):
                @plsc.parallel_loop(0, FB)
                def field(f_):
                    fa = f0base + f_

                    @plsc.parallel_loop(0, BH // LANES, unroll=4)
                    def bwin(w):
                        xx = xv[fa, pl.ds(w * LANES, LANES)]
                        t = jnp.clip((xx - MIN_VAL) * INV_STEP,
                                     0.0, float(N_BINS - 1))
                        g = t.astype(jnp.int32)
                        g = g + jnp.where(t > g.astype(jnp.float32), 1, 0)
                        bu = plsc.load_gather(
                            bounds_v, [jnp.minimum(g, N_BINS - 2)])
                        bd = plsc.load_gather(
                            bounds_v, [jnp.maximum(g - 1, 0)])
                        up = jnp.where((g < N_BINS - 1) & (bu < xx), 1, 0)
                        dn = jnp.where((g > 0) & (bd >= xx), 1, 0)
                        av = (g + up - dn) * (HIDDEN + 1)
                        bt_ = w // 8
                        bl_ = (w % 8) * LANES
                        for h in range(HIDDEN):
                            vals = plsc.load_gather(tv, [av + h])
                            vb[f_, h // 8, bt_, h % 8, pl.ds(bl_, LANES)] = vals

            def pair(p, c):
                f0 = p * 2 * FB

                @pl.when(p > 0)
                def _():
                    pltpu.make_async_copy(
                        vb0, out_hbm.at[pl.ds(0, FB), :, pl.ds(bt0, BH // 128)],
                        sem0).wait()

                fields_into(vb0, f0)
                pltpu.async_copy(
                    vb0, out_hbm.at[pl.ds(f0, FB), :, pl.ds(bt0, BH // 128)],
                    sem0)

                @pl.when(p > 0)
                def _():
                    pltpu.make_async_copy(
                        vb1, out_hbm.at[pl.ds(0, FB), :, pl.ds(bt0, BH // 128)],
                        sem1).wait()

                fields_into(vb1, f0 + FB)
                pltpu.async_copy(
                    vb1, out_hbm.at[pl.ds(f0 + FB, FB), :,
                                    pl.ds(bt0, BH // 128)],
                    sem1)
                return c

            lax.fori_loop(0, NPAIR, pair, 0, unroll=False)
            pltpu.make_async_copy(
                vb0, out_hbm.at[pl.ds(0, FB), :, pl.ds(bt0, BH // 128)],
                sem0).wait()
            pltpu.make_async_copy(
                vb1, out_hbm.at[pl.ds(0, FB), :, pl.ds(bt0, BH // 128)],
                sem1).wait()
            return carry

        lax.fori_loop(0, BW // BH, half, 0, unroll=False)

    return _sc_embed


def kernel(x, bounds, table):
    tpad = jnp.pad(table, ((0, 0), (0, 1)))
    raw = _build()(x.T, bounds, tpad.reshape(N_BINS * (HIDDEN + 1)))
    out = raw.transpose(2, 4, 0, 1, 3)
    return out.reshape(BATCH, FIELDS, HIDDEN)
